# baseline (speedup 1.0000x reference)
import functools

import jax
import jax.numpy as jnp
from jax import lax
from jax.experimental import pallas as pl
from jax.experimental.pallas import tpu as pltpu
from jax.experimental.pallas import tpu_sc as plsc

MAXLEN = 200
EMBED_DIM = 64
BATCH = 4096
ROW = MAXLEN * EMBED_DIM
LANES = 16
NUM_CORES = 2
NUM_SUBCORES = 16
NUM_WORKERS = NUM_CORES * NUM_SUBCORES
ROWS_PER_WORKER = BATCH // NUM_WORKERS
R = 2
CHUNKS = ROWS_PER_WORKER // R
SLICES = ROW // LANES

_mesh = plsc.VectorSubcoreMesh(core_axis_name="c", subcore_axis_name="s")


@functools.partial(
    pl.kernel,
    mesh=_mesh,
    out_type=jax.ShapeDtypeStruct((BATCH, ROW), jnp.float32),
    scratch_types=[
        pltpu.VMEM((ROW,), jnp.float32),
        pltpu.VMEM((R, ROW), jnp.float32),
    ],
)
def _sc_add(x_hbm, pos_hbm, out_hbm, pos_v, buf):
    wid = lax.axis_index("s") * NUM_CORES + lax.axis_index("c")
    base = wid * ROWS_PER_WORKER
    pltpu.sync_copy(pos_hbm, pos_v)

    def chunk_body(c, carry):
        row0 = base + c * R
        pltpu.sync_copy(x_hbm.at[pl.ds(row0, R)], buf)

        def j_body(j, carry2):
            off = j * LANES
            pv = pos_v[pl.ds(off, LANES)]
            for r in range(R):
                buf[r, pl.ds(off, LANES)] = buf[r, pl.ds(off, LANES)] + pv
            return carry2

        lax.fori_loop(0, SLICES, j_body, 0)
        pltpu.sync_copy(buf, out_hbm.at[pl.ds(row0, R)])
        return carry

    lax.fori_loop(0, CHUNKS, chunk_body, 0)


def kernel(x, pos_table):
    out = _sc_add(x.reshape(BATCH, ROW), pos_table.reshape(ROW))
    return out.reshape(BATCH, MAXLEN, EMBED_DIM)


# speedup vs baseline: 1.3265x; 1.3265x over previous
import functools

import jax
import jax.numpy as jnp
from jax import lax
from jax.experimental import pallas as pl
from jax.experimental.pallas import tpu as pltpu
from jax.experimental.pallas import tpu_sc as plsc

MAXLEN = 200
EMBED_DIM = 64
BATCH = 4096
ROW = MAXLEN * EMBED_DIM
LANES = 16
NUM_CORES = 2
NUM_SUBCORES = 16
NUM_WORKERS = NUM_CORES * NUM_SUBCORES
ROWS_PER_WORKER = BATCH // NUM_WORKERS
R = 2
NBUF = 2
CHUNKS = ROWS_PER_WORKER // R
N_OUTER = CHUNKS // NBUF
SLICES = ROW // LANES

_mesh = plsc.VectorSubcoreMesh(core_axis_name="c", subcore_axis_name="s")


@functools.partial(
    pl.kernel,
    mesh=_mesh,
    out_type=jax.ShapeDtypeStruct((BATCH, ROW), jnp.float32),
    scratch_types=[
        pltpu.VMEM((ROW,), jnp.float32),
        pltpu.VMEM((R, ROW), jnp.float32),
        pltpu.VMEM((R, ROW), jnp.float32),
        pltpu.VMEM((R, ROW), jnp.float32),
        pltpu.VMEM((R, ROW), jnp.float32),
        pltpu.SemaphoreType.DMA,
        pltpu.SemaphoreType.DMA,
        pltpu.SemaphoreType.DMA,
        pltpu.SemaphoreType.DMA,
    ],
)
def _sc_add(x_hbm, pos_hbm, out_hbm, pos_v, in0, in1, ou0, ou1, si0, si1, so0, so1):
    ins = (in0, in1)
    outs = (ou0, ou1)
    sis = (si0, si1)
    sos = (so0, so1)
    wid = lax.axis_index("s") * NUM_CORES + lax.axis_index("c")
    base = wid * ROWS_PER_WORKER
    pltpu.sync_copy(pos_hbm, pos_v)

    for b in range(NBUF):
        pltpu.async_copy(x_hbm.at[pl.ds(base + b * R, R)], ins[b], sis[b])

    def outer(i, carry):
        c0 = i * NBUF
        for b in range(NBUF):
            c = c0 + b
            row0 = base + c * R

            @pl.when(i > 0)
            def _():
                pltpu.make_async_copy(
                    outs[b], out_hbm.at[pl.ds(row0, R)], sos[b]
                ).wait()

            pltpu.make_async_copy(
                x_hbm.at[pl.ds(row0, R)], ins[b], sis[b]
            ).wait()

            def j_body(j, carry2):
                off = j * LANES
                pv = pos_v[pl.ds(off, LANES)]
                for r in range(R):
                    outs[b][r, pl.ds(off, LANES)] = (
                        ins[b][r, pl.ds(off, LANES)] + pv
                    )
                return carry2

            lax.fori_loop(0, SLICES, j_body, 0, unroll=2)

            @pl.when(i < N_OUTER - 1)
            def _():
                pltpu.async_copy(
                    x_hbm.at[pl.ds(row0 + NBUF * R, R)], ins[b], sis[b]
                )

            pltpu.async_copy(outs[b], out_hbm.at[pl.ds(row0, R)], sos[b])
        return carry

    lax.fori_loop(0, N_OUTER, outer, 0)

    for b in range(NBUF):
        row0 = base + (CHUNKS - NBUF + b) * R
        pltpu.make_async_copy(outs[b], out_hbm.at[pl.ds(row0, R)], sos[b]).wait()


def kernel(x, pos_table):
    out = _sc_add(x.reshape(BATCH, ROW), pos_table.reshape(ROW))
    return out.reshape(BATCH, MAXLEN, EMBED_DIM)


# speedup vs baseline: 1.7303x; 1.3044x over previous
import functools

import jax
import jax.numpy as jnp
from jax import lax
from jax.experimental import pallas as pl
from jax.experimental.pallas import tpu as pltpu
from jax.experimental.pallas import tpu_sc as plsc

MAXLEN = 200
EMBED_DIM = 64
BATCH = 4096
ROW = MAXLEN * EMBED_DIM
LANES = 16
NUM_CORES = 2
NUM_SUBCORES = 16
NUM_WORKERS = NUM_CORES * NUM_SUBCORES
ROWS_PER_WORKER = BATCH // NUM_WORKERS
R = 2
NBUF = 2
CHUNKS = ROWS_PER_WORKER // R
N_OUTER = CHUNKS // NBUF
SLICES = ROW // LANES

_mesh = plsc.VectorSubcoreMesh(core_axis_name="c", subcore_axis_name="s")


@functools.partial(
    pl.kernel,
    mesh=_mesh,
    out_type=jax.ShapeDtypeStruct((BATCH, ROW), jnp.float32),
    scratch_types=[
        pltpu.VMEM((ROW,), jnp.float32),
        pltpu.VMEM((R, ROW), jnp.float32),
        pltpu.VMEM((R, ROW), jnp.float32),
        pltpu.VMEM((R, ROW), jnp.float32),
        pltpu.VMEM((R, ROW), jnp.float32),
        pltpu.SemaphoreType.DMA,
        pltpu.SemaphoreType.DMA,
        pltpu.SemaphoreType.DMA,
        pltpu.SemaphoreType.DMA,
    ],
)
def _sc_add(x_hbm, pos_hbm, out_hbm, pos_v, in0, in1, ou0, ou1, si0, si1, so0, so1):
    ins = (in0, in1)
    outs = (ou0, ou1)
    sis = (si0, si1)
    sos = (so0, so1)
    wid = lax.axis_index("s") * NUM_CORES + lax.axis_index("c")
    base = wid * ROWS_PER_WORKER
    pltpu.sync_copy(pos_hbm, pos_v)

    for b in range(NBUF):
        pltpu.async_copy(x_hbm.at[pl.ds(base + b * R, R)], ins[b], sis[b])

    def outer(i, carry):
        c0 = i * NBUF
        for b in range(NBUF):
            c = c0 + b
            row0 = base + c * R

            @pl.when(i > 0)
            def _():
                pltpu.make_async_copy(
                    outs[b], out_hbm.at[pl.ds(row0, R)], sos[b]
                ).wait()

            pltpu.make_async_copy(
                x_hbm.at[pl.ds(row0, R)], ins[b], sis[b]
            ).wait()

            in_b = ins[b]
            out_b = outs[b]

            @plsc.parallel_loop(0, SLICES, 1, unroll=8)
            def _(j):
                off = j * LANES
                pv = pos_v[pl.ds(off, LANES)]
                for r in range(R):
                    out_b[r, pl.ds(off, LANES)] = in_b[r, pl.ds(off, LANES)] + pv

            @pl.when(i < N_OUTER - 1)
            def _():
                pltpu.async_copy(
                    x_hbm.at[pl.ds(row0 + NBUF * R, R)], ins[b], sis[b]
                )

            pltpu.async_copy(outs[b], out_hbm.at[pl.ds(row0, R)], sos[b])
        return carry

    lax.fori_loop(0, N_OUTER, outer, 0)

    for b in range(NBUF):
        row0 = base + (CHUNKS - NBUF + b) * R
        pltpu.make_async_copy(outs[b], out_hbm.at[pl.ds(row0, R)], sos[b]).wait()


def kernel(x, pos_table):
    out = _sc_add(x.reshape(BATCH, ROW), pos_table.reshape(ROW))
    return out.reshape(BATCH, MAXLEN, EMBED_DIM)


# speedup vs baseline: 4.4713x; 2.5841x over previous
import functools

import jax
import jax.numpy as jnp
from jax import lax
from jax.experimental import pallas as pl
from jax.experimental.pallas import tpu as pltpu
from jax.experimental.pallas import tpu_sc as plsc

MAXLEN = 200
EMBED_DIM = 64
BATCH = 4096
NROWS = MAXLEN * EMBED_DIM
LANES = 16
NUM_CORES = 2
NUM_SUBCORES = 16
NUM_WORKERS = NUM_CORES * NUM_SUBCORES
ROWS_PER_WORKER = NROWS // NUM_WORKERS
RCHUNK = 8
NCHUNK = ROWS_PER_WORKER // RCHUNK
VSLICES = BATCH // LANES
NBUF = 3

_mesh = plsc.VectorSubcoreMesh(core_axis_name="c", subcore_axis_name="s")


@functools.partial(
    pl.kernel,
    mesh=_mesh,
    out_type=jax.ShapeDtypeStruct((NROWS, BATCH), jnp.float32),
    scratch_types=[
        pltpu.VMEM((ROWS_PER_WORKER + LANES,), jnp.float32),
        pltpu.VMEM((RCHUNK, BATCH), jnp.float32),
        pltpu.VMEM((RCHUNK, BATCH), jnp.float32),
        pltpu.VMEM((RCHUNK, BATCH), jnp.float32),
        pltpu.SemaphoreType.DMA,
        pltpu.SemaphoreType.DMA,
        pltpu.SemaphoreType.DMA,
        pltpu.SemaphoreType.DMA,
        pltpu.SemaphoreType.DMA,
        pltpu.SemaphoreType.DMA,
    ],
)
def _sc_add(x_hbm, pos_hbm, out_hbm, pos_v, b0, b1, b2,
            si0, si1, si2, so0, so1, so2):
    bufs = (b0, b1, b2)
    sis = (si0, si1, si2)
    sos = (so0, so1, so2)
    wid = lax.axis_index("s") * NUM_CORES + lax.axis_index("c")
    base = wid * ROWS_PER_WORKER
    pltpu.sync_copy(pos_hbm.at[pl.ds(base, ROWS_PER_WORKER)],
                    pos_v.at[pl.ds(0, ROWS_PER_WORKER)])

    for b in range(NBUF):
        pltpu.async_copy(x_hbm.at[pl.ds(base + b * RCHUNK, RCHUNK)],
                         bufs[b], sis[b])

    def outer(i, carry):
        c0 = i * NBUF
        for b in range(NBUF):
            c = c0 + b

            @pl.when(c < NCHUNK)
            def _():
                row0 = base + c * RCHUNK
                buf = bufs[b]

                pltpu.make_async_copy(
                    x_hbm.at[pl.ds(row0, RCHUNK)], buf, sis[b]
                ).wait()

                pvec = pos_v[pl.ds(c * RCHUNK, LANES)]
                for r in range(RCHUNK):
                    pv = jnp.full((LANES,), pvec[r], jnp.float32)

                    @plsc.parallel_loop(0, VSLICES, 1, unroll=8)
                    def _(j):
                        buf[r, pl.ds(j * LANES, LANES)] = (
                            buf[r, pl.ds(j * LANES, LANES)] + pv
                        )

                pltpu.async_copy(buf, out_hbm.at[pl.ds(row0, RCHUNK)],
                                 sos[b])

                bn = (b + 1) % NBUF
                bufn = bufs[bn]

                @pl.when(jnp.logical_and(c >= NBUF - 1, c < NCHUNK - 1))
                def _():
                    rown = base + (c - (NBUF - 1)) * RCHUNK
                    pltpu.make_async_copy(
                        bufn, out_hbm.at[pl.ds(rown, RCHUNK)], sos[bn]
                    ).wait()
                    pltpu.async_copy(
                        x_hbm.at[pl.ds(base + (c + 1) * RCHUNK, RCHUNK)],
                        bufn, sis[bn],
                    )
        return carry

    lax.fori_loop(0, (NCHUNK + NBUF - 1) // NBUF, outer, 0)

    for k in range(NCHUNK - NBUF, NCHUNK):
        row0 = base + k * RCHUNK
        pltpu.make_async_copy(bufs[k % NBUF],
                              out_hbm.at[pl.ds(row0, RCHUNK)],
                              sos[k % NBUF]).wait()


def kernel(x, pos_table):
    x2 = jnp.transpose(x, (1, 2, 0)).reshape(NROWS, BATCH)
    pos_flat = pos_table.reshape(NROWS)
    out2 = _sc_add(x2, pos_flat)
    return jnp.transpose(out2.reshape(MAXLEN, EMBED_DIM, BATCH), (2, 0, 1))


# speedup vs baseline: 5.5151x; 1.2334x over previous
import functools

import jax
import jax.numpy as jnp
from jax import lax
from jax.experimental import pallas as pl
from jax.experimental.pallas import tpu as pltpu
from jax.experimental.pallas import tpu_sc as plsc

MAXLEN = 200
EMBED_DIM = 64
BATCH = 4096
NROWS = MAXLEN * EMBED_DIM
LANES = 16
NUM_CORES = 2
NUM_SUBCORES = 16
NUM_WORKERS = NUM_CORES * NUM_SUBCORES
ROWS_PER_WORKER = NROWS // NUM_WORKERS
RCHUNK = 4
NCHUNK = ROWS_PER_WORKER // RCHUNK
VSLICES = BATCH // LANES
NBUF = 2

_mesh = plsc.VectorSubcoreMesh(core_axis_name="c", subcore_axis_name="s")


@functools.partial(
    pl.kernel,
    mesh=_mesh,
    out_type=jax.ShapeDtypeStruct((NROWS, BATCH), jnp.float32),
    scratch_types=[
        pltpu.VMEM((ROWS_PER_WORKER + LANES,), jnp.float32),
        pltpu.VMEM((RCHUNK, BATCH), jnp.float32),
        pltpu.VMEM((RCHUNK, BATCH), jnp.float32),
        pltpu.VMEM((RCHUNK, BATCH), jnp.float32),
        pltpu.VMEM((RCHUNK, BATCH), jnp.float32),
        pltpu.SemaphoreType.DMA,
        pltpu.SemaphoreType.DMA,
        pltpu.SemaphoreType.DMA,
        pltpu.SemaphoreType.DMA,
    ],
)
def _sc_add(x_hbm, pos_hbm, out_hbm, pos_v, in0, in1, ou0, ou1,
            si0, si1, so0, so1):
    ins = (in0, in1)
    ous = (ou0, ou1)
    sis = (si0, si1)
    sos = (so0, so1)
    wid = lax.axis_index("s") * NUM_CORES + lax.axis_index("c")
    base = wid * ROWS_PER_WORKER
    pltpu.sync_copy(pos_hbm.at[pl.ds(base, ROWS_PER_WORKER)],
                    pos_v.at[pl.ds(0, ROWS_PER_WORKER)])

    for b in range(NBUF):
        pltpu.async_copy(x_hbm.at[pl.ds(base + b * RCHUNK, RCHUNK)],
                         ins[b], sis[b])

    def outer(i, carry):
        c0 = i * NBUF
        for b in range(NBUF):
            c = c0 + b
            row0 = base + c * RCHUNK
            in_b = ins[b]
            ou_b = ous[b]

            pltpu.make_async_copy(
                x_hbm.at[pl.ds(row0, RCHUNK)], in_b, sis[b]
            ).wait()

            @pl.when(i > 0)
            def _():
                pltpu.make_async_copy(
                    ou_b, out_hbm.at[pl.ds(row0, RCHUNK)], sos[b]
                ).wait()

            pvec = pos_v[pl.ds(c * RCHUNK, LANES)]
            for r in range(RCHUNK):
                pv = jnp.full((LANES,), pvec[r], jnp.float32)

                @plsc.parallel_loop(0, VSLICES, 1, unroll=8)
                def _(j):
                    ou_b[r, pl.ds(j * LANES, LANES)] = (
                        in_b[r, pl.ds(j * LANES, LANES)] + pv
                    )

            @pl.when(c < NCHUNK - NBUF)
            def _():
                pltpu.async_copy(
                    x_hbm.at[pl.ds(row0 + NBUF * RCHUNK, RCHUNK)],
                    in_b, sis[b],
                )

            pltpu.async_copy(ou_b, out_hbm.at[pl.ds(row0, RCHUNK)], sos[b])
        return carry

    lax.fori_loop(0, NCHUNK // NBUF, outer, 0)

    for k in range(NCHUNK - NBUF, NCHUNK):
        row0 = base + k * RCHUNK
        pltpu.make_async_copy(ous[k % NBUF],
                              out_hbm.at[pl.ds(row0, RCHUNK)],
                              sos[k % NBUF]).wait()


def kernel(x, pos_table):
    x2 = jnp.transpose(x, (1, 2, 0)).reshape(NROWS, BATCH)
    pos_flat = pos_table.reshape(NROWS)
    out2 = _sc_add(x2, pos_flat)
    return jnp.transpose(out2.reshape(MAXLEN, EMBED_DIM, BATCH), (2, 0, 1))
